# Initial kernel scaffold; baseline (speedup 1.0000x reference)
import jax
import jax.numpy as jnp
from jax.experimental import pallas as pl


def kernel(x, emb_weight):
    raise NotImplementedError("write your pallas kernel here")


# speedup vs baseline: 2.7465x; 2.7465x over previous
import jax
import jax.numpy as jnp
from jax.experimental import pallas as pl


def _copy_block(w_ref, o_ref):
    o_ref[...] = w_ref[...]


def kernel(x, emb_weight):
    del x
    S, D = emb_weight.shape
    BLOCK = 512
    return pl.pallas_call(
        _copy_block,
        grid=(S // BLOCK,),
        in_specs=[pl.BlockSpec((BLOCK, D), lambda i: (i, 0))],
        out_specs=pl.BlockSpec((BLOCK, D), lambda i: (i, 0)),
        out_shape=jax.ShapeDtypeStruct((S, D), emb_weight.dtype),
    )(emb_weight)
